# baseline (speedup 1.0000x reference)
import functools

import jax
import jax.numpy as jnp
from jax import lax
from jax.experimental import pallas as pl
from jax.experimental.pallas import tpu as pltpu
from jax.experimental.pallas import tpu_sc as plsc

NNODES = 10000
D = 128
DH = 64
KSTEPS = 10
ALPHA = 0.1

NCORES = 2
NSUB = 16
CH = 128
NCH = 160
EPT = NCH * CH
EPAD = NSUB * EPT
NROWS = 10112
NR2 = 2 * NROWS
ROWS_PER_SUB = NROWS // NSUB

_mesh = plsc.VectorSubcoreMesh(core_axis_name="c", subcore_axis_name="s",
                               num_cores=NCORES)


NG = 2
NGRP = NCH // NG


@functools.partial(
    pl.kernel,
    mesh=_mesh,
    compiler_params=pltpu.CompilerParams(use_tc_tiling_on_sc=False),
    out_type=jax.ShapeDtypeStruct((NR2, DH), jnp.float32),
    scratch_types=[
        pltpu.VMEM((NCH, CH), jnp.int32),
        pltpu.VMEM((NCH, CH), jnp.int32),
        [[pltpu.VMEM((CH, DH), jnp.float32) for _ in range(NG)]
         for _ in range(2)],
        pltpu.VMEM_SHARED((NROWS, DH), jnp.float32),
        pltpu.SemaphoreType.DMA,
        pltpu.SemaphoreType.DMA,
        pltpu.SemaphoreType.DMA,
        pltpu.SemaphoreType.DMA,
    ],
)
def _spmv(g_hbm, src_hbm, dst_hbm, zeros_hbm, out_hbm, src_v, dst_v, bufs,
          acc, sg0, sg1, ss0, ss1):
    cid = lax.axis_index("c")
    sid = lax.axis_index("s")

    pltpu.sync_copy(src_hbm.at[cid * NSUB + sid], src_v)
    pltpu.sync_copy(dst_hbm.at[sid], dst_v)

    sg = (sg0, sg1)
    ss = (ss0, ss1)

    def _fire_gathers(t, p):
        for b in range(NG):
            pltpu.async_copy(g_hbm.at[src_v.at[t * NG + b]], bufs[p][b], sg[p])

    def _wait_gathers(t, p):
        for b in range(NG):
            pltpu.make_async_copy(g_hbm.at[src_v.at[t * NG + b]], bufs[p][b],
                                  sg[p]).wait()

    def _fire_scatters(t, p):
        for b in range(NG):
            pltpu.async_copy(bufs[p][b], acc.at[dst_v.at[t * NG + b]], ss[p],
                             add=True)

    def _wait_scatters(t, p):
        for b in range(NG):
            pltpu.make_async_copy(bufs[p][b], acc.at[dst_v.at[t * NG + b]],
                                  ss[p]).wait()

    _fire_gathers(0, 0)
    pltpu.sync_copy(zeros_hbm,
                    acc.at[pl.ds(sid * ROWS_PER_SUB, ROWS_PER_SUB)])
    plsc.subcore_barrier()

    def _pair(tt, carry):
        t0 = tt * 2

        @pl.when(tt >= 1)
        def _():
            _wait_scatters(t0 - 1, 1)
        _wait_gathers(t0, 0)
        _fire_scatters(t0, 0)
        _fire_gathers(t0 + 1, 1)

        _wait_scatters(t0, 0)
        _wait_gathers(t0 + 1, 1)
        _fire_scatters(t0 + 1, 1)

        @pl.when(tt < NGRP // 2 - 1)
        def _():
            _fire_gathers(t0 + 2, 0)

        return carry

    lax.fori_loop(0, NGRP // 2, _pair, 0)
    _wait_scatters(NGRP - 1, 1)

    plsc.subcore_barrier()
    pltpu.sync_copy(
        acc.at[pl.ds(sid * ROWS_PER_SUB, ROWS_PER_SUB)],
        out_hbm.at[pl.ds(cid * NROWS + sid * ROWS_PER_SUB, ROWS_PER_SUB)])


BM = 1264


def _init_body(p_ref, x_ref, c2_ref, ax_ref, g0_ref, rinv_ref):
    deg = p_ref[...] + 1.0
    s = lax.rsqrt(deg)
    c2_ref[...] = (1.0 - ALPHA) / deg
    ax_ref[...] = ALPHA * s * x_ref[...]
    g0_ref[...] = s * x_ref[...]
    rinv_ref[...] = jnp.sqrt(deg)


_init = pl.pallas_call(
    _init_body,
    grid=(NR2 // BM,),
    in_specs=[pl.BlockSpec((BM, DH), lambda i: (i, 0))] * 2,
    out_specs=[pl.BlockSpec((BM, DH), lambda i: (i, 0))] * 4,
    out_shape=[jax.ShapeDtypeStruct((NR2, DH), jnp.float32)] * 4,
)


def _combine_body(p_ref, g_ref, c2_ref, ax_ref, o_ref):
    o_ref[...] = c2_ref[...] * (p_ref[...] + g_ref[...]) + ax_ref[...]


_combine = pl.pallas_call(
    _combine_body,
    grid=(NR2 // BM,),
    in_specs=[pl.BlockSpec((BM, DH), lambda i: (i, 0))] * 4,
    out_specs=pl.BlockSpec((BM, DH), lambda i: (i, 0)),
    out_shape=jax.ShapeDtypeStruct((NR2, DH), jnp.float32),
)

FM = 1000


def _final_body(gl_ref, gh_ref, rl_ref, rh_ref, wl_ref, wh_ref, b_ref, o_ref):
    hl = gl_ref[...] * rl_ref[...]
    hh = gh_ref[...] * rh_ref[...]
    dims = (((1,), (1,)), ((), ()))
    o_ref[...] = (
        lax.dot_general(hl, wl_ref[...], dims, preferred_element_type=jnp.float32)
        + lax.dot_general(hh, wh_ref[...], dims, preferred_element_type=jnp.float32)
        + b_ref[...])


_final = pl.pallas_call(
    _final_body,
    grid=(NNODES // FM,),
    in_specs=[
        pl.BlockSpec((FM, DH), lambda i: (i, 0)),
        pl.BlockSpec((FM, DH), lambda i: (i, 0)),
        pl.BlockSpec((FM, DH), lambda i: (i, 0)),
        pl.BlockSpec((FM, DH), lambda i: (i, 0)),
        pl.BlockSpec((D, DH), lambda i: (0, 0)),
        pl.BlockSpec((D, DH), lambda i: (0, 0)),
        pl.BlockSpec((1, D), lambda i: (0, 0)),
    ],
    out_specs=pl.BlockSpec((FM, D), lambda i: (i, 0)),
    out_shape=jax.ShapeDtypeStruct((NNODES, D), jnp.float32),
)


def kernel(x, edge_index, W, b):
    e = edge_index.shape[1]
    pad = EPAD - e
    order = jnp.argsort(edge_index[0])
    src = jnp.concatenate(
        [edge_index[0][order], jnp.zeros((pad,), jnp.int32)]).reshape(
            NSUB, NCH, CH)
    src2 = jnp.concatenate([src, src + NROWS])
    dst = jnp.concatenate(
        [edge_index[1][order], jnp.full((pad,), NNODES, jnp.int32)]).reshape(
            NSUB, NCH, CH)
    zeros = jnp.zeros((ROWS_PER_SUB, DH), jnp.float32)
    xp = jnp.pad(x, ((0, NROWS - NNODES), (0, 0)))
    x2 = jnp.concatenate([xp[:, :DH], xp[:, DH:]], axis=0)
    ones_g = jnp.ones((NR2, DH), jnp.float32)

    p = _spmv(ones_g, src2, dst, zeros)
    c2, ax, g, rinv = _init(p, x2)
    for _ in range(KSTEPS):
        p = _spmv(g, src2, dst, zeros)
        g = _combine(p, g, c2, ax)
    return _final(g[:NNODES], g[NROWS:NROWS + NNODES],
                  rinv[:NNODES], rinv[NROWS:NROWS + NNODES],
                  W[:, :DH], W[:, DH:], b.reshape(1, D))


# speedup vs baseline: 2.5300x; 2.5300x over previous
import functools

import jax
import jax.numpy as jnp
from jax import lax
from jax.experimental import pallas as pl
from jax.experimental.pallas import tpu as pltpu
from jax.experimental.pallas import tpu_sc as plsc

NNODES = 10000
D = 128
DH = 64
KSTEPS = 10
ALPHA = 0.1

NCORES = 2
NSUB = 16
CH = 128
NCH = 160
EPT = NCH * CH
EPAD = NSUB * EPT
NROWS = 10112
NR2 = 2 * NROWS
ROWS_PER_SUB = NROWS // NSUB

_mesh = plsc.VectorSubcoreMesh(core_axis_name="c", subcore_axis_name="s",
                               num_cores=NCORES)


NG = 2
NGRP = NCH // NG


GPS = 8
SGC = GPS * NG
NSG = NCH // SGC


@functools.partial(
    pl.kernel,
    mesh=_mesh,
    compiler_params=pltpu.CompilerParams(use_tc_tiling_on_sc=False),
    out_type=jax.ShapeDtypeStruct((NR2, DH), jnp.float32),
    scratch_types=[
        pltpu.VMEM((2, SGC, CH), jnp.int32),
        pltpu.VMEM((2, SGC, CH), jnp.int32),
        [[pltpu.VMEM((CH, DH), jnp.float32) for _ in range(NG)]
         for _ in range(2)],
        pltpu.VMEM_SHARED((NROWS, DH), jnp.float32),
        pltpu.VMEM_SHARED((NROWS, DH), jnp.float32),
        pltpu.SemaphoreType.DMA,
        pltpu.SemaphoreType.DMA,
        pltpu.SemaphoreType.DMA,
        pltpu.SemaphoreType.DMA,
        pltpu.SemaphoreType.DMA,
        pltpu.SemaphoreType.DMA,
    ],
)
def _spmv(g_hbm, src_hbm, dst_hbm, zeros_hbm, out_hbm, src_r, dst_r, bufs,
          gsp, acc, si0, si1, sg0, sg1, ss0, ss1):
    cid = lax.axis_index("c")
    sid = lax.axis_index("s")
    base = cid * NROWS + sid * ROWS_PER_SUB

    si = (si0, si1)
    sg = (sg0, sg1)
    ss = (ss0, ss1)

    def _start_sidx(S, s):
        pltpu.async_copy(src_hbm.at[sid, pl.ds(S * SGC, SGC)], src_r.at[s],
                         si[s])
        pltpu.async_copy(dst_hbm.at[sid, pl.ds(S * SGC, SGC)], dst_r.at[s],
                         si[s])

    def _wait_sidx(S, s):
        pltpu.make_async_copy(src_hbm.at[sid, pl.ds(S * SGC, SGC)],
                              src_r.at[s], si[s]).wait()
        pltpu.make_async_copy(dst_hbm.at[sid, pl.ds(S * SGC, SGC)],
                              dst_r.at[s], si[s]).wait()

    def _fire_gathers(s, t, p):
        for b in range(NG):
            pltpu.async_copy(gsp.at[src_r.at[s, t * NG + b]], bufs[p][b],
                             sg[p])

    def _wait_gathers(s, t, p):
        for b in range(NG):
            pltpu.make_async_copy(gsp.at[src_r.at[s, t * NG + b]], bufs[p][b],
                                  sg[p]).wait()

    def _fire_scatters(s, t, p):
        for b in range(NG):
            pltpu.async_copy(bufs[p][b], acc.at[dst_r.at[s, t * NG + b]],
                             ss[p], add=True)

    def _wait_scatters(s, t, p):
        for b in range(NG):
            pltpu.make_async_copy(bufs[p][b], acc.at[dst_r.at[s, t * NG + b]],
                                  ss[p]).wait()

    _start_sidx(0, 0)
    pltpu.sync_copy(g_hbm.at[pl.ds(base, ROWS_PER_SUB)],
                    gsp.at[pl.ds(sid * ROWS_PER_SUB, ROWS_PER_SUB)])
    pltpu.sync_copy(zeros_hbm,
                    acc.at[pl.ds(sid * ROWS_PER_SUB, ROWS_PER_SUB)])
    plsc.subcore_barrier()

    for S in range(NSG):
        s = S % 2
        _wait_sidx(S, s)
        if S + 1 < NSG:
            _start_sidx(S + 1, 1 - s)
        _fire_gathers(s, 0, 0)

        def _pair(tt, carry, s=s):
            t0 = tt * 2

            @pl.when(tt >= 1)
            def _():
                _wait_scatters(s, t0 - 1, 1)
            _wait_gathers(s, t0, 0)
            _fire_scatters(s, t0, 0)
            _fire_gathers(s, t0 + 1, 1)

            _wait_scatters(s, t0, 0)
            _wait_gathers(s, t0 + 1, 1)
            _fire_scatters(s, t0 + 1, 1)

            @pl.when(tt < GPS // 2 - 1)
            def _():
                _fire_gathers(s, t0 + 2, 0)

            return carry

        lax.fori_loop(0, GPS // 2, _pair, 0)
        _wait_scatters(s, GPS - 1, 1)

    plsc.subcore_barrier()
    pltpu.sync_copy(
        acc.at[pl.ds(sid * ROWS_PER_SUB, ROWS_PER_SUB)],
        out_hbm.at[pl.ds(cid * NROWS + sid * ROWS_PER_SUB, ROWS_PER_SUB)])


BM = 1264


def _init_body(p_ref, x_ref, c2_ref, ax_ref, g0_ref, rinv_ref):
    deg = p_ref[...] + 1.0
    s = lax.rsqrt(deg)
    c2_ref[...] = (1.0 - ALPHA) / deg
    ax_ref[...] = ALPHA * s * x_ref[...]
    g0_ref[...] = s * x_ref[...]
    rinv_ref[...] = jnp.sqrt(deg)


_init = pl.pallas_call(
    _init_body,
    grid=(NR2 // BM,),
    in_specs=[pl.BlockSpec((BM, DH), lambda i: (i, 0))] * 2,
    out_specs=[pl.BlockSpec((BM, DH), lambda i: (i, 0))] * 4,
    out_shape=[jax.ShapeDtypeStruct((NR2, DH), jnp.float32)] * 4,
)


def _combine_body(p_ref, g_ref, c2_ref, ax_ref, o_ref):
    o_ref[...] = c2_ref[...] * (p_ref[...] + g_ref[...]) + ax_ref[...]


_combine = pl.pallas_call(
    _combine_body,
    grid=(NR2 // BM,),
    in_specs=[pl.BlockSpec((BM, DH), lambda i: (i, 0))] * 4,
    out_specs=pl.BlockSpec((BM, DH), lambda i: (i, 0)),
    out_shape=jax.ShapeDtypeStruct((NR2, DH), jnp.float32),
)

FM = 1000


def _final_body(gl_ref, gh_ref, rl_ref, rh_ref, wl_ref, wh_ref, b_ref, o_ref):
    hl = gl_ref[...] * rl_ref[...]
    hh = gh_ref[...] * rh_ref[...]
    dims = (((1,), (1,)), ((), ()))
    o_ref[...] = (
        lax.dot_general(hl, wl_ref[...], dims, preferred_element_type=jnp.float32)
        + lax.dot_general(hh, wh_ref[...], dims, preferred_element_type=jnp.float32)
        + b_ref[...])


_final = pl.pallas_call(
    _final_body,
    grid=(NNODES // FM,),
    in_specs=[
        pl.BlockSpec((FM, DH), lambda i: (i, 0)),
        pl.BlockSpec((FM, DH), lambda i: (i, 0)),
        pl.BlockSpec((FM, DH), lambda i: (i, 0)),
        pl.BlockSpec((FM, DH), lambda i: (i, 0)),
        pl.BlockSpec((D, DH), lambda i: (0, 0)),
        pl.BlockSpec((D, DH), lambda i: (0, 0)),
        pl.BlockSpec((1, D), lambda i: (0, 0)),
    ],
    out_specs=pl.BlockSpec((FM, D), lambda i: (i, 0)),
    out_shape=jax.ShapeDtypeStruct((NNODES, D), jnp.float32),
)


def kernel(x, edge_index, W, b):
    e = edge_index.shape[1]
    pad = EPAD - e
    src = jnp.concatenate(
        [edge_index[0], jnp.zeros((pad,), jnp.int32)]).reshape(NSUB, NCH, CH)
    dst = jnp.concatenate(
        [edge_index[1], jnp.full((pad,), NNODES, jnp.int32)]).reshape(
            NSUB, NCH, CH)
    zeros = jnp.zeros((ROWS_PER_SUB, DH), jnp.float32)
    xp = jnp.pad(x, ((0, NROWS - NNODES), (0, 0)))
    x2 = jnp.concatenate([xp[:, :DH], xp[:, DH:]], axis=0)
    ones_g = jnp.ones((NR2, DH), jnp.float32)

    p = _spmv(ones_g, src, dst, zeros)
    c2, ax, g, rinv = _init(p, x2)
    for _ in range(KSTEPS):
        p = _spmv(g, src, dst, zeros)
        g = _combine(p, g, c2, ax)
    return _final(g[:NNODES], g[NROWS:NROWS + NNODES],
                  rinv[:NNODES], rinv[NROWS:NROWS + NNODES],
                  W[:, :DH], W[:, DH:], b.reshape(1, D))


# speedup vs baseline: 2.6508x; 1.0477x over previous
import functools

import jax
import jax.numpy as jnp
from jax import lax
from jax.experimental import pallas as pl
from jax.experimental.pallas import tpu as pltpu
from jax.experimental.pallas import tpu_sc as plsc

NNODES = 10000
D = 128
DH = 64
KSTEPS = 10
ALPHA = 0.1

NCORES = 2
NSUB = 16
CH = 128
NCH = 160
EPT = NCH * CH
EPAD = NSUB * EPT
NROWS = 10112
NR2 = 2 * NROWS
ROWS_PER_SUB = NROWS // NSUB

_mesh = plsc.VectorSubcoreMesh(core_axis_name="c", subcore_axis_name="s",
                               num_cores=NCORES)


NG = 2
NGRP = NCH // NG


GPS = 8
SGC = GPS * NG
NSG = NCH // SGC


@functools.partial(
    pl.kernel,
    mesh=_mesh,
    compiler_params=pltpu.CompilerParams(use_tc_tiling_on_sc=False),
    out_type=jax.ShapeDtypeStruct((NR2, DH), jnp.float32),
    scratch_types=[
        pltpu.VMEM((2, SGC, CH), jnp.int32),
        pltpu.VMEM((2, SGC, CH), jnp.int32),
        [[pltpu.VMEM((CH, DH), jnp.float32) for _ in range(NG)]
         for _ in range(2)],
        pltpu.VMEM_SHARED((NROWS, DH), jnp.float32),
        pltpu.VMEM_SHARED((NROWS, DH), jnp.float32),
        pltpu.SemaphoreType.DMA,
        pltpu.SemaphoreType.DMA,
        pltpu.SemaphoreType.DMA,
        pltpu.SemaphoreType.DMA,
        pltpu.SemaphoreType.DMA,
        pltpu.SemaphoreType.DMA,
    ],
)
def _spmv(g_hbm, src_hbm, dst_hbm, out_hbm, src_r, dst_r, bufs,
          gsp, acc, si0, si1, sg0, sg1, ss0, ss1):
    cid = lax.axis_index("c")
    sid = lax.axis_index("s")
    base = cid * NROWS + sid * ROWS_PER_SUB

    si = (si0, si1)
    sg = (sg0, sg1)
    ss = (ss0, ss1)

    def _start_sidx(S, s):
        pltpu.async_copy(src_hbm.at[sid, pl.ds(S * SGC, SGC)], src_r.at[s],
                         si[s])
        pltpu.async_copy(dst_hbm.at[sid, pl.ds(S * SGC, SGC)], dst_r.at[s],
                         si[s])

    def _wait_sidx(S, s):
        pltpu.make_async_copy(src_hbm.at[sid, pl.ds(S * SGC, SGC)],
                              src_r.at[s], si[s]).wait()
        pltpu.make_async_copy(dst_hbm.at[sid, pl.ds(S * SGC, SGC)],
                              dst_r.at[s], si[s]).wait()

    def _fire_gathers(s, t, p):
        for b in range(NG):
            pltpu.async_copy(gsp.at[src_r.at[s, t * NG + b]], bufs[p][b],
                             sg[p])

    def _wait_gathers(s, t, p):
        for b in range(NG):
            pltpu.make_async_copy(gsp.at[src_r.at[s, t * NG + b]], bufs[p][b],
                                  sg[p]).wait()

    def _fire_scatters(s, t, p):
        for b in range(NG):
            pltpu.async_copy(bufs[p][b], acc.at[dst_r.at[s, t * NG + b]],
                             ss[p], add=True)

    def _wait_scatters(s, t, p):
        for b in range(NG):
            pltpu.make_async_copy(bufs[p][b], acc.at[dst_r.at[s, t * NG + b]],
                                  ss[p]).wait()

    _start_sidx(0, 0)
    pltpu.sync_copy(g_hbm.at[pl.ds(base, ROWS_PER_SUB)],
                    gsp.at[pl.ds(sid * ROWS_PER_SUB, ROWS_PER_SUB)])
    pltpu.sync_copy(g_hbm.at[pl.ds(base, ROWS_PER_SUB)],
                    acc.at[pl.ds(sid * ROWS_PER_SUB, ROWS_PER_SUB)])
    plsc.subcore_barrier()

    for S in range(NSG):
        s = S % 2
        _wait_sidx(S, s)
        if S + 1 < NSG:
            _start_sidx(S + 1, 1 - s)
        _fire_gathers(s, 0, 0)

        def _pair(tt, carry, s=s):
            t0 = tt * 2

            @pl.when(tt >= 1)
            def _():
                _wait_scatters(s, t0 - 1, 1)
            _wait_gathers(s, t0, 0)
            _fire_scatters(s, t0, 0)
            _fire_gathers(s, t0 + 1, 1)

            _wait_scatters(s, t0, 0)
            _wait_gathers(s, t0 + 1, 1)
            _fire_scatters(s, t0 + 1, 1)

            @pl.when(tt < GPS // 2 - 1)
            def _():
                _fire_gathers(s, t0 + 2, 0)

            return carry

        lax.fori_loop(0, GPS // 2, _pair, 0)
        _wait_scatters(s, GPS - 1, 1)

    plsc.subcore_barrier()
    pltpu.sync_copy(
        acc.at[pl.ds(sid * ROWS_PER_SUB, ROWS_PER_SUB)],
        out_hbm.at[pl.ds(cid * NROWS + sid * ROWS_PER_SUB, ROWS_PER_SUB)])


@functools.partial(
    pl.kernel,
    mesh=_mesh,
    compiler_params=pltpu.CompilerParams(use_tc_tiling_on_sc=False),
    out_type=jax.ShapeDtypeStruct((NR2, DH), jnp.float32),
    scratch_types=[
        pltpu.VMEM((NCH, CH), jnp.int32),
        pltpu.VMEM((CH, DH), jnp.float32),
        pltpu.VMEM_SHARED((NROWS, DH), jnp.float32),
        pltpu.SemaphoreType.DMA,
        pltpu.SemaphoreType.DMA,
    ],
)
def _deg(dst_hbm, zeros_hbm, ones_hbm, out_hbm, dst_v, ones_v, acc, ss0, ss1):
    cid = lax.axis_index("c")
    sid = lax.axis_index("s")

    pltpu.sync_copy(dst_hbm.at[sid], dst_v)
    pltpu.sync_copy(ones_hbm, ones_v)
    pltpu.sync_copy(zeros_hbm,
                    acc.at[pl.ds(sid * ROWS_PER_SUB, ROWS_PER_SUB)])
    plsc.subcore_barrier()

    ss = (ss0, ss1)

    def _fire(t, p):
        pltpu.async_copy(ones_v, acc.at[dst_v.at[t]], ss[p], add=True)

    def _wait(t, p):
        pltpu.make_async_copy(ones_v, acc.at[dst_v.at[t]], ss[p]).wait()

    def _pair(j, carry):
        t0 = j * 2

        @pl.when(j >= 1)
        def _():
            _wait(t0 - 2, 0)
            _wait(t0 - 1, 1)
        _fire(t0, 0)
        _fire(t0 + 1, 1)
        return carry

    lax.fori_loop(0, NCH // 2, _pair, 0)
    _wait(NCH - 2, 0)
    _wait(NCH - 1, 1)

    plsc.subcore_barrier()
    pltpu.sync_copy(
        acc.at[pl.ds(sid * ROWS_PER_SUB, ROWS_PER_SUB)],
        out_hbm.at[pl.ds(cid * NROWS + sid * ROWS_PER_SUB, ROWS_PER_SUB)])


BM = 1264


def _init_body(p_ref, x_ref, c2_ref, ax_ref, g0_ref, rinv_ref):
    deg = p_ref[...] + 1.0
    s = lax.rsqrt(deg)
    c2_ref[...] = (1.0 - ALPHA) / deg
    ax_ref[...] = ALPHA * s * x_ref[...]
    g0_ref[...] = s * x_ref[...]
    rinv_ref[...] = jnp.sqrt(deg)


_init = pl.pallas_call(
    _init_body,
    grid=(NR2 // BM,),
    in_specs=[pl.BlockSpec((BM, DH), lambda i: (i, 0))] * 2,
    out_specs=[pl.BlockSpec((BM, DH), lambda i: (i, 0))] * 4,
    out_shape=[jax.ShapeDtypeStruct((NR2, DH), jnp.float32)] * 4,
)


def _combine_body(p_ref, c2_ref, ax_ref, o_ref):
    o_ref[...] = c2_ref[...] * p_ref[...] + ax_ref[...]


_combine = pl.pallas_call(
    _combine_body,
    grid=(NR2 // BM,),
    in_specs=[pl.BlockSpec((BM, DH), lambda i: (i, 0))] * 3,
    out_specs=pl.BlockSpec((BM, DH), lambda i: (i, 0)),
    out_shape=jax.ShapeDtypeStruct((NR2, DH), jnp.float32),
)

FM = 1000


def _final_body(gl_ref, gh_ref, rl_ref, rh_ref, wl_ref, wh_ref, b_ref, o_ref):
    hl = gl_ref[...] * rl_ref[...]
    hh = gh_ref[...] * rh_ref[...]
    dims = (((1,), (1,)), ((), ()))
    o_ref[...] = (
        lax.dot_general(hl, wl_ref[...], dims, preferred_element_type=jnp.float32)
        + lax.dot_general(hh, wh_ref[...], dims, preferred_element_type=jnp.float32)
        + b_ref[...])


_final = pl.pallas_call(
    _final_body,
    grid=(NNODES // FM,),
    in_specs=[
        pl.BlockSpec((FM, DH), lambda i: (i, 0)),
        pl.BlockSpec((FM, DH), lambda i: (i, 0)),
        pl.BlockSpec((FM, DH), lambda i: (i, 0)),
        pl.BlockSpec((FM, DH), lambda i: (i, 0)),
        pl.BlockSpec((D, DH), lambda i: (0, 0)),
        pl.BlockSpec((D, DH), lambda i: (0, 0)),
        pl.BlockSpec((1, D), lambda i: (0, 0)),
    ],
    out_specs=pl.BlockSpec((FM, D), lambda i: (i, 0)),
    out_shape=jax.ShapeDtypeStruct((NNODES, D), jnp.float32),
)


def kernel(x, edge_index, W, b):
    e = edge_index.shape[1]
    pad = EPAD - e
    src = jnp.concatenate(
        [edge_index[0], jnp.zeros((pad,), jnp.int32)]).reshape(NSUB, NCH, CH)
    dst = jnp.concatenate(
        [edge_index[1], jnp.full((pad,), NNODES, jnp.int32)]).reshape(
            NSUB, NCH, CH)
    zeros = jnp.zeros((ROWS_PER_SUB, DH), jnp.float32)
    ones = jnp.ones((CH, DH), jnp.float32)
    xp = jnp.pad(x, ((0, NROWS - NNODES), (0, 0)))
    x2 = jnp.concatenate([xp[:, :DH], xp[:, DH:]], axis=0)

    p = _deg(dst, zeros, ones)
    c2, ax, g, rinv = _init(p, x2)
    for _ in range(KSTEPS):
        p = _spmv(g, src, dst)
        g = _combine(p, c2, ax)
    return _final(g[:NNODES], g[NROWS:NROWS + NNODES],
                  rinv[:NNODES], rinv[NROWS:NROWS + NNODES],
                  W[:, :DH], W[:, DH:], b.reshape(1, D))


# speedup vs baseline: 2.7599x; 1.0412x over previous
import functools

import jax
import jax.numpy as jnp
from jax import lax
from jax.experimental import pallas as pl
from jax.experimental.pallas import tpu as pltpu
from jax.experimental.pallas import tpu_sc as plsc

NNODES = 10000
D = 128
DH = 64
KSTEPS = 10
ALPHA = 0.1

NCORES = 2
NSUB = 16
CH = 128
NCH = 160
EPT = NCH * CH
EPAD = NSUB * EPT
NROWS = 10112
NR2 = 2 * NROWS
ROWS_PER_SUB = NROWS // NSUB

_mesh = plsc.VectorSubcoreMesh(core_axis_name="c", subcore_axis_name="s",
                               num_cores=NCORES)


NG = 2
NGRP = NCH // NG


GPS = 16
SGC = GPS * NG
NSG = NCH // SGC


@functools.partial(
    pl.kernel,
    mesh=_mesh,
    compiler_params=pltpu.CompilerParams(use_tc_tiling_on_sc=False),
    out_type=jax.ShapeDtypeStruct((NR2, DH), jnp.float32),
    scratch_types=[
        pltpu.VMEM((2, SGC, CH), jnp.int32),
        pltpu.VMEM((2, SGC, CH), jnp.int32),
        [[pltpu.VMEM((CH, DH), jnp.float32) for _ in range(NG)]
         for _ in range(2)],
        pltpu.VMEM_SHARED((NROWS, DH), jnp.float32),
        pltpu.VMEM_SHARED((NROWS, DH), jnp.float32),
        pltpu.SemaphoreType.DMA,
        pltpu.SemaphoreType.DMA,
        pltpu.SemaphoreType.DMA,
        pltpu.SemaphoreType.DMA,
        pltpu.SemaphoreType.DMA,
        pltpu.SemaphoreType.DMA,
    ],
)
def _spmv(g_hbm, src_hbm, dst_hbm, out_hbm, src_r, dst_r, bufs,
          gsp, acc, si0, si1, sg0, sg1, ss0, ss1):
    cid = lax.axis_index("c")
    sid = lax.axis_index("s")
    base = cid * NROWS + sid * ROWS_PER_SUB

    si = (si0, si1)
    sg = (sg0, sg1)
    ss = (ss0, ss1)

    def _start_sidx(S, s):
        pltpu.async_copy(src_hbm.at[sid, pl.ds(S * SGC, SGC)], src_r.at[s],
                         si[s])
        pltpu.async_copy(dst_hbm.at[sid, pl.ds(S * SGC, SGC)], dst_r.at[s],
                         si[s])

    def _wait_sidx(S, s):
        pltpu.make_async_copy(src_hbm.at[sid, pl.ds(S * SGC, SGC)],
                              src_r.at[s], si[s]).wait()
        pltpu.make_async_copy(dst_hbm.at[sid, pl.ds(S * SGC, SGC)],
                              dst_r.at[s], si[s]).wait()

    def _fire_gathers(s, t, p):
        for b in range(NG):
            pltpu.async_copy(gsp.at[src_r.at[s, t * NG + b]], bufs[p][b],
                             sg[p])

    def _wait_gathers(s, t, p):
        for b in range(NG):
            pltpu.make_async_copy(gsp.at[src_r.at[s, t * NG + b]], bufs[p][b],
                                  sg[p]).wait()

    def _fire_scatters(s, t, p):
        for b in range(NG):
            pltpu.async_copy(bufs[p][b], acc.at[dst_r.at[s, t * NG + b]],
                             ss[p], add=True)

    def _wait_scatters(s, t, p):
        for b in range(NG):
            pltpu.make_async_copy(bufs[p][b], acc.at[dst_r.at[s, t * NG + b]],
                                  ss[p]).wait()

    _start_sidx(0, 0)
    pltpu.sync_copy(g_hbm.at[pl.ds(base, ROWS_PER_SUB)],
                    gsp.at[pl.ds(sid * ROWS_PER_SUB, ROWS_PER_SUB)])
    pltpu.sync_copy(g_hbm.at[pl.ds(base, ROWS_PER_SUB)],
                    acc.at[pl.ds(sid * ROWS_PER_SUB, ROWS_PER_SUB)])
    plsc.subcore_barrier()

    for S in range(NSG):
        s = S % 2
        _wait_sidx(S, s)
        if S + 1 < NSG:
            _start_sidx(S + 1, 1 - s)
        _fire_gathers(s, 0, 0)

        def _pair(tt, carry, s=s):
            t0 = tt * 2

            @pl.when(tt >= 1)
            def _():
                _wait_scatters(s, t0 - 1, 1)
            _wait_gathers(s, t0, 0)
            _fire_scatters(s, t0, 0)
            _fire_gathers(s, t0 + 1, 1)

            _wait_scatters(s, t0, 0)
            _wait_gathers(s, t0 + 1, 1)
            _fire_scatters(s, t0 + 1, 1)

            @pl.when(tt < GPS // 2 - 1)
            def _():
                _fire_gathers(s, t0 + 2, 0)

            return carry

        lax.fori_loop(0, GPS // 2, _pair, 0)
        _wait_scatters(s, GPS - 1, 1)

    plsc.subcore_barrier()
    pltpu.sync_copy(
        acc.at[pl.ds(sid * ROWS_PER_SUB, ROWS_PER_SUB)],
        out_hbm.at[pl.ds(cid * NROWS + sid * ROWS_PER_SUB, ROWS_PER_SUB)])


@functools.partial(
    pl.kernel,
    mesh=_mesh,
    compiler_params=pltpu.CompilerParams(use_tc_tiling_on_sc=False),
    out_type=jax.ShapeDtypeStruct((NR2, DH), jnp.float32),
    scratch_types=[
        pltpu.VMEM((NCH, CH), jnp.int32),
        pltpu.VMEM((CH, DH), jnp.float32),
        pltpu.VMEM_SHARED((NROWS, DH), jnp.float32),
        pltpu.SemaphoreType.DMA,
        pltpu.SemaphoreType.DMA,
    ],
)
def _deg(dst_hbm, zeros_hbm, ones_hbm, out_hbm, dst_v, ones_v, acc, ss0, ss1):
    cid = lax.axis_index("c")
    sid = lax.axis_index("s")

    pltpu.sync_copy(dst_hbm.at[sid], dst_v)
    pltpu.sync_copy(ones_hbm, ones_v)
    pltpu.sync_copy(zeros_hbm,
                    acc.at[pl.ds(sid * ROWS_PER_SUB, ROWS_PER_SUB)])
    plsc.subcore_barrier()

    ss = (ss0, ss1)

    def _fire(t, p):
        pltpu.async_copy(ones_v, acc.at[dst_v.at[t]], ss[p], add=True)

    def _wait(t, p):
        pltpu.make_async_copy(ones_v, acc.at[dst_v.at[t]], ss[p]).wait()

    def _pair(j, carry):
        t0 = j * 2

        @pl.when(j >= 1)
        def _():
            _wait(t0 - 2, 0)
            _wait(t0 - 1, 1)
        _fire(t0, 0)
        _fire(t0 + 1, 1)
        return carry

    lax.fori_loop(0, NCH // 2, _pair, 0)
    _wait(NCH - 2, 0)
    _wait(NCH - 1, 1)

    plsc.subcore_barrier()
    pltpu.sync_copy(
        acc.at[pl.ds(sid * ROWS_PER_SUB, ROWS_PER_SUB)],
        out_hbm.at[pl.ds(cid * NROWS + sid * ROWS_PER_SUB, ROWS_PER_SUB)])


BM = 1264


def _init_body(p_ref, x_ref, c2_ref, ax_ref, g0_ref, rinv_ref):
    deg = p_ref[...] + 1.0
    s = lax.rsqrt(deg)
    c2_ref[...] = (1.0 - ALPHA) / deg
    ax_ref[...] = ALPHA * s * x_ref[...]
    g0_ref[...] = s * x_ref[...]
    rinv_ref[...] = jnp.sqrt(deg)


_init = pl.pallas_call(
    _init_body,
    grid=(NR2 // BM,),
    in_specs=[pl.BlockSpec((BM, DH), lambda i: (i, 0))] * 2,
    out_specs=[pl.BlockSpec((BM, DH), lambda i: (i, 0))] * 4,
    out_shape=[jax.ShapeDtypeStruct((NR2, DH), jnp.float32)] * 4,
)


def _combine_body(p_ref, c2_ref, ax_ref, o_ref):
    o_ref[...] = c2_ref[...] * p_ref[...] + ax_ref[...]


_combine = pl.pallas_call(
    _combine_body,
    grid=(NR2 // BM,),
    in_specs=[pl.BlockSpec((BM, DH), lambda i: (i, 0))] * 3,
    out_specs=pl.BlockSpec((BM, DH), lambda i: (i, 0)),
    out_shape=jax.ShapeDtypeStruct((NR2, DH), jnp.float32),
)

FM = 1000


def _final_body(gl_ref, gh_ref, rl_ref, rh_ref, wl_ref, wh_ref, b_ref, o_ref):
    hl = gl_ref[...] * rl_ref[...]
    hh = gh_ref[...] * rh_ref[...]
    dims = (((1,), (1,)), ((), ()))
    o_ref[...] = (
        lax.dot_general(hl, wl_ref[...], dims, preferred_element_type=jnp.float32)
        + lax.dot_general(hh, wh_ref[...], dims, preferred_element_type=jnp.float32)
        + b_ref[...])


_final = pl.pallas_call(
    _final_body,
    grid=(NNODES // FM,),
    in_specs=[
        pl.BlockSpec((FM, DH), lambda i: (i, 0)),
        pl.BlockSpec((FM, DH), lambda i: (i, 0)),
        pl.BlockSpec((FM, DH), lambda i: (i, 0)),
        pl.BlockSpec((FM, DH), lambda i: (i, 0)),
        pl.BlockSpec((D, DH), lambda i: (0, 0)),
        pl.BlockSpec((D, DH), lambda i: (0, 0)),
        pl.BlockSpec((1, D), lambda i: (0, 0)),
    ],
    out_specs=pl.BlockSpec((FM, D), lambda i: (i, 0)),
    out_shape=jax.ShapeDtypeStruct((NNODES, D), jnp.float32),
)


def kernel(x, edge_index, W, b):
    e = edge_index.shape[1]
    pad = EPAD - e
    src = jnp.concatenate(
        [edge_index[0], jnp.zeros((pad,), jnp.int32)]).reshape(NSUB, NCH, CH)
    dst = jnp.concatenate(
        [edge_index[1], jnp.full((pad,), NNODES, jnp.int32)]).reshape(
            NSUB, NCH, CH)
    zeros = jnp.zeros((ROWS_PER_SUB, DH), jnp.float32)
    ones = jnp.ones((CH, DH), jnp.float32)
    xp = jnp.pad(x, ((0, NROWS - NNODES), (0, 0)))
    x2 = jnp.concatenate([xp[:, :DH], xp[:, DH:]], axis=0)

    p = _deg(dst, zeros, ones)
    c2, ax, g, rinv = _init(p, x2)
    for _ in range(KSTEPS):
        p = _spmv(g, src, dst)
        g = _combine(p, c2, ax)
    return _final(g[:NNODES], g[NROWS:NROWS + NNODES],
                  rinv[:NNODES], rinv[NROWS:NROWS + NNODES],
                  W[:, :DH], W[:, DH:], b.reshape(1, D))


# speedup vs baseline: 2.7602x; 1.0001x over previous
import functools

import jax
import jax.numpy as jnp
from jax import lax
from jax.experimental import pallas as pl
from jax.experimental.pallas import tpu as pltpu
from jax.experimental.pallas import tpu_sc as plsc

NNODES = 10000
D = 128
DH = 64
KSTEPS = 10
ALPHA = 0.1

NCORES = 2
NSUB = 16
CH = 128
NCH = 160
EPT = NCH * CH
EPAD = NSUB * EPT
NROWS = 10112
NR2 = 2 * NROWS
ROWS_PER_SUB = NROWS // NSUB

_mesh = plsc.VectorSubcoreMesh(core_axis_name="c", subcore_axis_name="s",
                               num_cores=NCORES)


NG = 2
GPS = 16
SGC = GPS * NG
NSG = NCH // SGC


@functools.partial(
    pl.kernel,
    mesh=_mesh,
    compiler_params=pltpu.CompilerParams(use_tc_tiling_on_sc=False),
    out_type=jax.ShapeDtypeStruct((NR2, DH), jnp.float32),
    scratch_types=[
        pltpu.VMEM((2, SGC, CH), jnp.int32),
        pltpu.VMEM((2, SGC, CH), jnp.int32),
        [[pltpu.VMEM((CH, DH), jnp.float32) for _ in range(NG)]
         for _ in range(2)],
        pltpu.VMEM_SHARED((NROWS, DH), jnp.float32),
        pltpu.VMEM_SHARED((NROWS, DH), jnp.float32),
        pltpu.SemaphoreType.DMA,
        pltpu.SemaphoreType.DMA,
        pltpu.SemaphoreType.DMA,
        pltpu.SemaphoreType.DMA,
        pltpu.SemaphoreType.DMA,
        pltpu.SemaphoreType.DMA,
    ],
)
def _spmv(g_hbm, src_hbm, dst_hbm, out_hbm, src_r, dst_r, bufs,
          gsp, acc, si0, si1, sg0, sg1, ss0, ss1):
    cid = lax.axis_index("c")
    sid = lax.axis_index("s")
    base = cid * NROWS + sid * ROWS_PER_SUB

    si = (si0, si1)
    sg = (sg0, sg1)
    ss = (ss0, ss1)

    def _start_sidx(S, s):
        pltpu.async_copy(src_hbm.at[sid, pl.ds(S * SGC, SGC)], src_r.at[s],
                         si[s])
        pltpu.async_copy(dst_hbm.at[sid, pl.ds(S * SGC, SGC)], dst_r.at[s],
                         si[s])

    def _wait_sidx(S, s):
        pltpu.make_async_copy(src_hbm.at[sid, pl.ds(S * SGC, SGC)],
                              src_r.at[s], si[s]).wait()
        pltpu.make_async_copy(dst_hbm.at[sid, pl.ds(S * SGC, SGC)],
                              dst_r.at[s], si[s]).wait()

    def _fire_gathers(s, t, p):
        for b in range(NG):
            pltpu.async_copy(gsp.at[src_r.at[s, t * NG + b]], bufs[p][b],
                             sg[p])

    def _wait_gathers(s, t, p):
        for b in range(NG):
            pltpu.make_async_copy(gsp.at[src_r.at[s, t * NG + b]], bufs[p][b],
                                  sg[p]).wait()

    def _fire_scatters(s, t, p):
        for b in range(NG):
            pltpu.async_copy(bufs[p][b], acc.at[dst_r.at[s, t * NG + b]],
                             ss[p], add=True)

    def _wait_scatters(s, t, p):
        for b in range(NG):
            pltpu.make_async_copy(bufs[p][b], acc.at[dst_r.at[s, t * NG + b]],
                                  ss[p]).wait()

    _start_sidx(0, 0)
    pltpu.sync_copy(g_hbm.at[pl.ds(base, ROWS_PER_SUB)],
                    gsp.at[pl.ds(sid * ROWS_PER_SUB, ROWS_PER_SUB)])
    pltpu.sync_copy(g_hbm.at[pl.ds(base, ROWS_PER_SUB)],
                    acc.at[pl.ds(sid * ROWS_PER_SUB, ROWS_PER_SUB)])
    plsc.subcore_barrier()

    for S in range(NSG):
        s = S % 2
        _wait_sidx(S, s)
        if S + 1 < NSG:
            _start_sidx(S + 1, 1 - s)
        _fire_gathers(s, 0, 0)

        def _pair(tt, carry, s=s):
            t0 = tt * 2

            @pl.when(tt >= 1)
            def _():
                _wait_scatters(s, t0 - 1, 1)
            _wait_gathers(s, t0, 0)
            _fire_scatters(s, t0, 0)
            _fire_gathers(s, t0 + 1, 1)

            _wait_scatters(s, t0, 0)
            _wait_gathers(s, t0 + 1, 1)
            _fire_scatters(s, t0 + 1, 1)

            @pl.when(tt < GPS // 2 - 1)
            def _():
                _fire_gathers(s, t0 + 2, 0)

            return carry

        lax.fori_loop(0, GPS // 2, _pair, 0)
        _wait_scatters(s, GPS - 1, 1)

    plsc.subcore_barrier()
    pltpu.sync_copy(
        acc.at[pl.ds(sid * ROWS_PER_SUB, ROWS_PER_SUB)],
        out_hbm.at[pl.ds(cid * NROWS + sid * ROWS_PER_SUB, ROWS_PER_SUB)])


@functools.partial(
    pl.kernel,
    mesh=_mesh,
    compiler_params=pltpu.CompilerParams(use_tc_tiling_on_sc=False),
    out_type=jax.ShapeDtypeStruct((NR2, DH), jnp.float32),
    scratch_types=[
        pltpu.VMEM((NCH, CH), jnp.int32),
        pltpu.VMEM((CH, DH), jnp.float32),
        pltpu.VMEM_SHARED((NROWS, DH), jnp.float32),
        pltpu.SemaphoreType.DMA,
        pltpu.SemaphoreType.DMA,
    ],
)
def _deg(dst_hbm, zeros_hbm, ones_hbm, out_hbm, dst_v, ones_v, acc, ss0, ss1):
    cid = lax.axis_index("c")
    sid = lax.axis_index("s")

    pltpu.sync_copy(dst_hbm.at[sid], dst_v)
    pltpu.sync_copy(ones_hbm, ones_v)
    pltpu.sync_copy(zeros_hbm,
                    acc.at[pl.ds(sid * ROWS_PER_SUB, ROWS_PER_SUB)])
    plsc.subcore_barrier()

    ss = (ss0, ss1)

    def _fire(t, p):
        pltpu.async_copy(ones_v, acc.at[dst_v.at[t]], ss[p], add=True)

    def _wait(t, p):
        pltpu.make_async_copy(ones_v, acc.at[dst_v.at[t]], ss[p]).wait()

    def _pair(j, carry):
        t0 = j * 2

        @pl.when(j >= 1)
        def _():
            _wait(t0 - 2, 0)
            _wait(t0 - 1, 1)
        _fire(t0, 0)
        _fire(t0 + 1, 1)
        return carry

    lax.fori_loop(0, NCH // 2, _pair, 0)
    _wait(NCH - 2, 0)
    _wait(NCH - 1, 1)

    plsc.subcore_barrier()
    pltpu.sync_copy(
        acc.at[pl.ds(sid * ROWS_PER_SUB, ROWS_PER_SUB)],
        out_hbm.at[pl.ds(cid * NROWS + sid * ROWS_PER_SUB, ROWS_PER_SUB)])


BM = 1264


def _init_body(p_ref, x_ref, c2_ref, ax_ref, g0_ref, rinv_ref):
    deg = p_ref[...] + 1.0
    s = lax.rsqrt(deg)
    c2_ref[...] = (1.0 - ALPHA) / deg
    ax_ref[...] = ALPHA * s * x_ref[...]
    g0_ref[...] = s * x_ref[...]
    rinv_ref[...] = jnp.sqrt(deg)


_init = pl.pallas_call(
    _init_body,
    grid=(NR2 // BM,),
    in_specs=[pl.BlockSpec((BM, DH), lambda i: (i, 0))] * 2,
    out_specs=[pl.BlockSpec((BM, DH), lambda i: (i, 0))] * 4,
    out_shape=[jax.ShapeDtypeStruct((NR2, DH), jnp.float32)] * 4,
)


def _combine_body(p_ref, c2_ref, ax_ref, o_ref):
    o_ref[...] = c2_ref[...] * p_ref[...] + ax_ref[...]


_combine = pl.pallas_call(
    _combine_body,
    grid=(NR2 // BM,),
    in_specs=[pl.BlockSpec((BM, DH), lambda i: (i, 0))] * 3,
    out_specs=pl.BlockSpec((BM, DH), lambda i: (i, 0)),
    out_shape=jax.ShapeDtypeStruct((NR2, DH), jnp.float32),
)

FM = 1000


def _final_body(gl_ref, gh_ref, rl_ref, rh_ref, wl_ref, wh_ref, b_ref, o_ref):
    hl = gl_ref[...] * rl_ref[...]
    hh = gh_ref[...] * rh_ref[...]
    dims = (((1,), (1,)), ((), ()))
    o_ref[...] = (
        lax.dot_general(hl, wl_ref[...], dims, preferred_element_type=jnp.float32)
        + lax.dot_general(hh, wh_ref[...], dims, preferred_element_type=jnp.float32)
        + b_ref[...])


_final = pl.pallas_call(
    _final_body,
    grid=(NNODES // FM,),
    in_specs=[
        pl.BlockSpec((FM, DH), lambda i: (i, 0)),
        pl.BlockSpec((FM, DH), lambda i: (i, 0)),
        pl.BlockSpec((FM, DH), lambda i: (i, 0)),
        pl.BlockSpec((FM, DH), lambda i: (i, 0)),
        pl.BlockSpec((D, DH), lambda i: (0, 0)),
        pl.BlockSpec((D, DH), lambda i: (0, 0)),
        pl.BlockSpec((1, D), lambda i: (0, 0)),
    ],
    out_specs=pl.BlockSpec((FM, D), lambda i: (i, 0)),
    out_shape=jax.ShapeDtypeStruct((NNODES, D), jnp.float32),
)


def kernel(x, edge_index, W, b):
    e = edge_index.shape[1]
    pad = EPAD - e
    src = jnp.concatenate(
        [edge_index[0], jnp.zeros((pad,), jnp.int32)]).reshape(NSUB, NCH, CH)
    dst = jnp.concatenate(
        [edge_index[1], jnp.full((pad,), NNODES, jnp.int32)]).reshape(
            NSUB, NCH, CH)
    zeros = jnp.zeros((ROWS_PER_SUB, DH), jnp.float32)
    ones = jnp.ones((CH, DH), jnp.float32)
    xp = jnp.pad(x, ((0, NROWS - NNODES), (0, 0)))
    x2 = jnp.concatenate([xp[:, :DH], xp[:, DH:]], axis=0)

    p = _deg(dst, zeros, ones)
    c2, ax, g, rinv = _init(p, x2)
    for _ in range(KSTEPS):
        p = _spmv(g, src, dst)
        g = _combine(p, c2, ax)
    return _final(g[:NNODES], g[NROWS:NROWS + NNODES],
                  rinv[:NNODES], rinv[NROWS:NROWS + NNODES],
                  W[:, :DH], W[:, DH:], b.reshape(1, D))
